# baseline (speedup 1.0000x reference)
import jax
import jax.numpy as jnp
from jax import lax
from jax.experimental import pallas as pl
from jax.experimental.pallas import tpu as pltpu
from jax.experimental.pallas import tpu_sc as plsc

B = 16384
EMB = 100
PITCH = 128
NFX = 56
NC, NS = 2, 16
NW = NC * NS
BPW = B // NW
CH = 128
NCH = BPW // CH
H1, H2 = 128, 32
BLK = 16384
BLKU = 16384


def _repack_body(in_ref, out_ref):
    out_ref[:, :EMB] = in_ref[:].T


def _repack(tab_t):
    n = tab_t.shape[1]
    grid = (pl.cdiv(n, BLKU),)
    return pl.pallas_call(
        _repack_body,
        grid=grid,
        in_specs=[pl.BlockSpec((EMB, BLKU), lambda i: (0, i))],
        out_specs=pl.BlockSpec((BLKU, PITCH), lambda i: (i, 0)),
        out_shape=jax.ShapeDtypeStruct((n, PITCH), jnp.float32),
        compiler_params=pltpu.CompilerParams(dimension_semantics=("arbitrary",)),
    )(tab_t)


def _gather_body(idx_ref, tab_ref, rows_out, idx_v, b0, b1, b2, b3, gsem, osem):
    bufs = (b0, b1, b2, b3)
    wid = lax.axis_index("s") * NC + lax.axis_index("c")
    base = wid * BPW
    pltpu.sync_copy(idx_ref.at[pl.ds(wid * NCH, NCH)], idx_v)
    gs = [pltpu.async_copy(tab_ref.at[idx_v.at[k]], bufs[k], gsem)
          for k in range(NCH)]
    os = []
    for k in range(NCH):
        gs[k].wait()
        os.append(pltpu.async_copy(
            bufs[k], rows_out.at[pl.ds(base + k * CH, CH)], osem))
    for o in os:
        o.wait()


def _make_gather(n):
    return pl.kernel(
        _gather_body,
        out_type=jax.ShapeDtypeStruct((B, PITCH), jnp.float32),
        mesh=plsc.VectorSubcoreMesh(core_axis_name="c", subcore_axis_name="s",
                                    num_cores=NC, num_subcores=NS),
        scratch_types=[
            pltpu.VMEM((NCH, CH), jnp.int32),
            pltpu.VMEM((CH, PITCH), jnp.float32),
            pltpu.VMEM((CH, PITCH), jnp.float32),
            pltpu.VMEM((CH, PITCH), jnp.float32),
            pltpu.VMEM((CH, PITCH), jnp.float32),
            pltpu.SemaphoreType.DMA,
            pltpu.SemaphoreType.DMA,
        ],
        name=n,
    )


_gather_u = _make_gather("gather_users")
_gather_a = _make_gather("gather_animes")


def _mlp_body(u_ref, a_ref, xt_ref, w1u_ref, w1a_ref, w1f_ref, b1_ref,
              w2_ref, b2_ref, w3_ref, b3_ref, o_ref):
    ft = xt_ref[2:, :]
    h = (jnp.dot(u_ref[:, :EMB], w1u_ref[:], preferred_element_type=jnp.float32)
         + jnp.dot(a_ref[:, :EMB], w1a_ref[:], preferred_element_type=jnp.float32)
         + lax.dot_general(ft, w1f_ref[:], (((0,), (0,)), ((), ())),
                           preferred_element_type=jnp.float32)
         + b1_ref[:])
    h = jnp.maximum(h, 0.0)
    h2 = jnp.dot(h, w2_ref[:], preferred_element_type=jnp.float32) + b2_ref[:]
    h2 = jnp.maximum(h2, 0.0)
    z = jnp.sum(h2 * w3_ref[:], axis=1, keepdims=True) + b3_ref[:]
    o_ref[:] = jax.nn.sigmoid(z)


_mlp = pl.pallas_call(
    _mlp_body,
    grid=(B // BLK,),
    in_specs=[
        pl.BlockSpec((BLK, PITCH), lambda i: (i, 0)),
        pl.BlockSpec((BLK, PITCH), lambda i: (i, 0)),
        pl.BlockSpec((NFX, BLK), lambda i: (0, i)),
        pl.BlockSpec((EMB, H1), lambda i: (0, 0)),
        pl.BlockSpec((EMB, H1), lambda i: (0, 0)),
        pl.BlockSpec((NFX - 2, H1), lambda i: (0, 0)),
        pl.BlockSpec((1, H1), lambda i: (0, 0)),
        pl.BlockSpec((H1, H2), lambda i: (0, 0)),
        pl.BlockSpec((1, H2), lambda i: (0, 0)),
        pl.BlockSpec((1, H2), lambda i: (0, 0)),
        pl.BlockSpec((1, 1), lambda i: (0, 0)),
    ],
    out_specs=pl.BlockSpec((BLK, 1), lambda i: (i, 0)),
    out_shape=jax.ShapeDtypeStruct((B, 1), jnp.float32),
    compiler_params=pltpu.CompilerParams(dimension_semantics=("arbitrary",)),
)


def kernel(x, users, animes, W1, b1, W2, b2, W3, b3):
    uidx = x[:, 0].astype(jnp.int32).reshape(B // CH, CH)
    aidx = x[:, 1].astype(jnp.int32).reshape(B // CH, CH)
    animes_p = _repack(animes.T)
    arows = _gather_a(aidx, animes_p)
    users_p = _repack(users.T)
    urows = _gather_u(uidx, users_p)
    w1u = W1[:, :EMB].T
    w1a = W1[:, EMB:2 * EMB].T
    w1f = W1[:, 2 * EMB:].T
    return _mlp(urows, arows, x.T, w1u, w1a, w1f, b1.reshape(1, H1),
                W2.T, b2.reshape(1, H2), W3, b3.reshape(1, 1))


# speedup vs baseline: 1.0345x; 1.0345x over previous
import jax
import jax.numpy as jnp
from jax import lax
from jax.experimental import pallas as pl
from jax.experimental.pallas import tpu as pltpu
from jax.experimental.pallas import tpu_sc as plsc

B = 16384
EMB = 100
PITCH = 128
NFX = 56
NC, NS = 2, 16
NW = NC * NS
BPW = B // NW
CH = 128
NCH = BPW // CH
H1, H2 = 128, 32
BLK = 8192
BLKU = 16384


def _repack_body(in_ref, out_ref):
    out_ref[:, :EMB] = in_ref[:].T


def _repack(tab_t):
    n = tab_t.shape[1]
    grid = (pl.cdiv(n, BLKU),)
    return pl.pallas_call(
        _repack_body,
        grid=grid,
        in_specs=[pl.BlockSpec((EMB, BLKU), lambda i: (0, i))],
        out_specs=pl.BlockSpec((BLKU, PITCH), lambda i: (i, 0)),
        out_shape=jax.ShapeDtypeStruct((n, PITCH), jnp.float32),
        compiler_params=pltpu.CompilerParams(dimension_semantics=("arbitrary",)),
    )(tab_t)


def _gather_body(idx_ref, tab_ref, rows_out, idx_v, b0, b1, b2, b3, gsem, osem):
    bufs = (b0, b1, b2, b3)
    wid = lax.axis_index("s") * NC + lax.axis_index("c")
    base = wid * BPW
    pltpu.sync_copy(idx_ref.at[pl.ds(wid * NCH, NCH)], idx_v)
    gs = [pltpu.async_copy(tab_ref.at[idx_v.at[k]], bufs[k], gsem)
          for k in range(NCH)]
    os = []
    for k in range(NCH):
        gs[k].wait()
        os.append(pltpu.async_copy(
            bufs[k], rows_out.at[pl.ds(base + k * CH, CH)], osem))
    for o in os:
        o.wait()


def _make_gather(n):
    return pl.kernel(
        _gather_body,
        out_type=jax.ShapeDtypeStruct((B, PITCH), jnp.float32),
        mesh=plsc.VectorSubcoreMesh(core_axis_name="c", subcore_axis_name="s",
                                    num_cores=NC, num_subcores=NS),
        scratch_types=[
            pltpu.VMEM((NCH, CH), jnp.int32),
            pltpu.VMEM((CH, PITCH), jnp.float32),
            pltpu.VMEM((CH, PITCH), jnp.float32),
            pltpu.VMEM((CH, PITCH), jnp.float32),
            pltpu.VMEM((CH, PITCH), jnp.float32),
            pltpu.SemaphoreType.DMA,
            pltpu.SemaphoreType.DMA,
        ],
        name=n,
    )


_gather_u = _make_gather("gather_users")
_gather_a = _make_gather("gather_animes")


def _mlp_body(u_ref, a_ref, xt_ref, w1u_ref, w1a_ref, w1f_ref, b1_ref,
              w2_ref, b2_ref, w3_ref, b3_ref, o_ref):
    ft = xt_ref[2:, :]
    h = (jnp.dot(u_ref[:, :EMB], w1u_ref[:], preferred_element_type=jnp.float32)
         + jnp.dot(a_ref[:, :EMB], w1a_ref[:], preferred_element_type=jnp.float32)
         + lax.dot_general(ft, w1f_ref[:], (((0,), (0,)), ((), ())),
                           preferred_element_type=jnp.float32)
         + b1_ref[:])
    h = jnp.maximum(h, 0.0)
    h2 = jnp.dot(h, w2_ref[:], preferred_element_type=jnp.float32) + b2_ref[:]
    h2 = jnp.maximum(h2, 0.0)
    z = jnp.sum(h2 * w3_ref[:], axis=1, keepdims=True) + b3_ref[:]
    o_ref[:] = jax.nn.sigmoid(z)


_mlp = pl.pallas_call(
    _mlp_body,
    grid=(B // BLK,),
    in_specs=[
        pl.BlockSpec((BLK, PITCH), lambda i: (i, 0)),
        pl.BlockSpec((BLK, PITCH), lambda i: (i, 0)),
        pl.BlockSpec((NFX, BLK), lambda i: (0, i)),
        pl.BlockSpec((EMB, H1), lambda i: (0, 0)),
        pl.BlockSpec((EMB, H1), lambda i: (0, 0)),
        pl.BlockSpec((NFX - 2, H1), lambda i: (0, 0)),
        pl.BlockSpec((1, H1), lambda i: (0, 0)),
        pl.BlockSpec((H1, H2), lambda i: (0, 0)),
        pl.BlockSpec((1, H2), lambda i: (0, 0)),
        pl.BlockSpec((1, H2), lambda i: (0, 0)),
        pl.BlockSpec((1, 1), lambda i: (0, 0)),
    ],
    out_specs=pl.BlockSpec((BLK, 1), lambda i: (i, 0)),
    out_shape=jax.ShapeDtypeStruct((B, 1), jnp.float32),
    compiler_params=pltpu.CompilerParams(dimension_semantics=("arbitrary",)),
)


def kernel(x, users, animes, W1, b1, W2, b2, W3, b3):
    uidx = x[:, 0].astype(jnp.int32).reshape(B // CH, CH)
    aidx = x[:, 1].astype(jnp.int32).reshape(B // CH, CH)
    animes_p = _repack(animes.T)
    arows = _gather_a(aidx, animes_p)
    users_p = _repack(users.T)
    urows = _gather_u(uidx, users_p)
    w1u = W1[:, :EMB].T
    w1a = W1[:, EMB:2 * EMB].T
    w1f = W1[:, 2 * EMB:].T
    return _mlp(urows, arows, x.T, w1u, w1a, w1f, b1.reshape(1, H1),
                W2.T, b2.reshape(1, H2), W3, b3.reshape(1, 1))
